# baseline (speedup 1.0000x reference)
import functools

import jax
import jax.numpy as jnp
from jax import lax
from jax.experimental import pallas as pl
from jax.experimental.pallas import tpu as pltpu
from jax.experimental.pallas import tpu_sc as plsc

N = 10000
D = 64
E = 640000
NC = 2
NS = 16
NW = NC * NS

CH = 128
NROW = E // CH
RPW_P = 160
NROW_P = RPW_P * NW
KSUB = 5
NG = RPW_P // KSUB

N_P = N + 8

CHK = 80
STRIPE = 640
STRIPE_LAST = N - (NS - 1) * STRIPE
assert STRIPE % CHK == 0 and STRIPE_LAST % CHK == 0


def _deg_body(dst_hbm, ones_hbm, zeros1_hbm, deg_out0, deg_out1,
              deg_sh, idx_v, ones_v, zeros_v, zbuf, ssem0, ssem1):
    c = lax.axis_index("c")
    s = lax.axis_index("s")
    wid = c * NS + s

    base = s * STRIPE

    @pl.when(s < NS - 1)
    def _():
        pltpu.sync_copy(zeros1_hbm, zbuf)
        pltpu.sync_copy(zbuf, deg_sh.at[pl.ds(base, STRIPE)])

    @pl.when(s == NS - 1)
    def _():
        pltpu.sync_copy(zeros1_hbm.at[pl.ds(0, STRIPE_LAST)],
                        zbuf.at[pl.ds(0, STRIPE_LAST)])
        pltpu.sync_copy(zbuf.at[pl.ds(0, STRIPE_LAST)],
                        deg_sh.at[pl.ds(base, STRIPE_LAST)])

    pltpu.sync_copy(ones_hbm, ones_v)
    pltpu.sync_copy(zeros1_hbm.at[pl.ds(0, CH)], zeros_v)
    plsc.subcore_barrier()

    row0 = wid * RPW_P
    ssems = (ssem0, ssem1)

    def fire(b, g, ssem):
        r = row0 + g * KSUB
        pltpu.sync_copy(dst_hbm.at[pl.ds(r, KSUB)], idx_v.at[b])
        real = r < NROW

        @pl.when(real)
        def _():
            for j in range(KSUB):
                pltpu.async_copy(ones_v, deg_sh.at[idx_v.at[b, j]], ssem,
                                 add=True)

        @pl.when(jnp.logical_not(real))
        def _():
            for j in range(KSUB):
                pltpu.async_copy(zeros_v, deg_sh.at[idx_v.at[b, j]], ssem,
                                 add=True)

    def drain(b, ssem):
        for j in range(KSUB):
            pltpu.make_async_copy(ones_v, deg_sh.at[idx_v.at[b, j]], ssem).wait()

    def pair(t, carry):
        for b in (0, 1):
            g = 2 * t + b

            @pl.when(t >= 1)
            def _():
                drain(b, ssems[b])

            fire(b, g, ssems[b])
        return carry

    lax.fori_loop(0, NG // 2, pair, 0)
    drain(0, ssem0)
    drain(1, ssem1)
    plsc.subcore_barrier()

    @pl.when(c == 0)
    def _():
        _flush_deg(s, deg_sh, zbuf, deg_out0)

    @pl.when(c == 1)
    def _():
        _flush_deg(s, deg_sh, zbuf, deg_out1)


def _flush_deg(s, deg_sh, zbuf, out_ref):
    base = s * STRIPE

    @pl.when(s < NS - 1)
    def _():
        pltpu.sync_copy(deg_sh.at[pl.ds(base, STRIPE)], zbuf)
        pltpu.sync_copy(zbuf, out_ref.at[pl.ds(base, STRIPE)])

    @pl.when(s == NS - 1)
    def _():
        pltpu.sync_copy(deg_sh.at[pl.ds(base, STRIPE_LAST)],
                        zbuf.at[pl.ds(0, STRIPE_LAST)])
        pltpu.sync_copy(zbuf.at[pl.ds(0, STRIPE_LAST)],
                        out_ref.at[pl.ds(base, STRIPE_LAST)])


def _deg_call(dst_r, ones_c, zeros1):
    mesh = plsc.VectorSubcoreMesh(core_axis_name="c", subcore_axis_name="s")
    return pl.kernel(
        _deg_body,
        out_type=[jax.ShapeDtypeStruct((N,), jnp.float32),
                  jax.ShapeDtypeStruct((N,), jnp.float32)],
        mesh=mesh,
        scratch_types=[
            pltpu.VMEM_SHARED((N,), jnp.float32),
            pltpu.VMEM((2, KSUB, CH), jnp.int32),
            pltpu.VMEM((CH,), jnp.float32),
            pltpu.VMEM((CH,), jnp.float32),
            pltpu.VMEM((STRIPE,), jnp.float32),
            pltpu.SemaphoreType.DMA,
            pltpu.SemaphoreType.DMA,
        ],
        compiler_params=pltpu.CompilerParams(use_tc_tiling_on_sc=False),
    )(dst_r, ones_c, zeros1)


def _agg_body(g_hbm, sd_hbm, zeros2_hbm, s_out0, s_out1,
              s_sh, idx_v, rows_v, zbuf,
              gsem0, gsem1, ssem0, ssem1):
    c = lax.axis_index("c")
    s = lax.axis_index("s")
    wid = c * NS + s

    _stage_table(s, zeros2_hbm, zbuf, s_sh, to_spmem=True)
    plsc.subcore_barrier()

    row0 = wid * RPW_P
    gsems = (gsem0, gsem1)
    ssems = (ssem0, ssem1)

    def load_and_fire_gathers(b, g, gsem):
        r = row0 + g * KSUB
        pltpu.sync_copy(sd_hbm.at[pl.ds(r, KSUB)], idx_v.at[b])
        for j in range(KSUB):
            pltpu.async_copy(g_hbm.at[idx_v.at[b, j, 0]], rows_v.at[b, j], gsem)

    def drain_gathers(b, gsem):
        for j in range(KSUB):
            pltpu.make_async_copy(g_hbm.at[idx_v.at[b, j, 0]],
                                  rows_v.at[b, j], gsem).wait()

    def fire_scatters(b, ssem):
        for j in range(KSUB):
            pltpu.async_copy(rows_v.at[b, j], s_sh.at[idx_v.at[b, j, 1]],
                             ssem, add=True)

    def drain_scatters(b, ssem):
        for j in range(KSUB):
            pltpu.make_async_copy(rows_v.at[b, j],
                                  s_sh.at[idx_v.at[b, j, 1]], ssem).wait()

    load_and_fire_gathers(0, 0, gsem0)

    def pair(t, carry):
        @pl.when(t >= 1)
        def _():
            drain_scatters(1, ssem1)

        load_and_fire_gathers(1, 2 * t + 1, gsem1)
        drain_gathers(0, gsem0)
        fire_scatters(0, ssem0)

        @pl.when(t < NG // 2 - 1)
        def _():
            drain_scatters(0, ssem0)
            load_and_fire_gathers(0, 2 * t + 2, gsem0)

        drain_gathers(1, gsem1)
        fire_scatters(1, ssem1)
        return carry

    lax.fori_loop(0, NG // 2, pair, 0)
    drain_scatters(0, ssem0)
    drain_scatters(1, ssem1)
    plsc.subcore_barrier()

    @pl.when(c == 0)
    def _():
        _stage_table(s, s_out0, zbuf, s_sh, to_spmem=False)

    @pl.when(c == 1)
    def _():
        _stage_table(s, s_out1, zbuf, s_sh, to_spmem=False)


def _stage_table(s, hbm_ref, zbuf, sh_ref, to_spmem):
    base = s * STRIPE

    def move_chunk(off):
        if to_spmem:
            pltpu.sync_copy(hbm_ref, zbuf)
            pltpu.sync_copy(zbuf, sh_ref.at[pl.ds(base + off, CHK)])
        else:
            pltpu.sync_copy(sh_ref.at[pl.ds(base + off, CHK)], zbuf)
            pltpu.sync_copy(zbuf, hbm_ref.at[pl.ds(base + off, CHK)])

    @pl.when(s < NS - 1)
    def _():
        for i in range(STRIPE // CHK):
            move_chunk(i * CHK)

    @pl.when(s == NS - 1)
    def _():
        for i in range(STRIPE_LAST // CHK):
            move_chunk(i * CHK)


def _agg_call(g, sd_r, zeros2):
    mesh = plsc.VectorSubcoreMesh(core_axis_name="c", subcore_axis_name="s")
    return pl.kernel(
        _agg_body,
        out_type=[jax.ShapeDtypeStruct((N, D), jnp.float32),
                  jax.ShapeDtypeStruct((N, D), jnp.float32)],
        mesh=mesh,
        scratch_types=[
            pltpu.VMEM_SHARED((N, D), jnp.float32),
            pltpu.VMEM((2, KSUB, 2, CH), jnp.int32),
            pltpu.VMEM((2, KSUB, CH, D), jnp.float32),
            pltpu.VMEM((CHK, D), jnp.float32),
            pltpu.SemaphoreType.DMA,
            pltpu.SemaphoreType.DMA,
            pltpu.SemaphoreType.DMA,
            pltpu.SemaphoreType.DMA,
        ],
        compiler_params=pltpu.CompilerParams(use_tc_tiling_on_sc=False),
    )(g, sd_r, zeros2)


def _scale_body(deg0_ref, deg1_ref, x_ref, g_ref, dinv_ref):
    total = deg0_ref[...] + deg1_ref[...] + 1.0
    dinv = lax.rsqrt(total)
    dinv_ref[...] = dinv
    g_ref[pl.ds(0, N), :] = x_ref[...] * dinv
    g_ref[pl.ds(N, N_P - N), :] = jnp.zeros((N_P - N, D), jnp.float32)


def _scale_call(deg0, deg1, x):
    return pl.pallas_call(
        _scale_body,
        out_shape=[
            jax.ShapeDtypeStruct((N_P, D), jnp.float32),
            jax.ShapeDtypeStruct((N, 1), jnp.float32),
        ],
    )(deg0, deg1, x)


def _z_body(s0_ref, s1_ref, g_ref, dinv_ref, w_ref, b_ref, z_ref):
    agg = (s0_ref[...] + s1_ref[...] + g_ref[pl.ds(0, N), :]) * dinv_ref[...]
    z = jnp.dot(agg, w_ref[...], preferred_element_type=jnp.float32) + b_ref[...]
    z_ref[...] = jnp.maximum(z, 0.0)


def _z_call(s0, s1, g, dinv, W, b2):
    return pl.pallas_call(
        _z_body,
        out_shape=jax.ShapeDtypeStruct((N, D), jnp.float32),
    )(s0, s1, g, dinv, W, b2)


BM = 2000
BN = 2048


def _gram_body(zi_ref, zj_ref, out_ref):
    out_ref[...] = lax.dot_general(
        zi_ref[...], zj_ref[...],
        (((1,), (1,)), ((), ())),
        preferred_element_type=jnp.float32,
    )


def _gram_call(z):
    grid = (N // BM, (N + BN - 1) // BN)
    return pl.pallas_call(
        _gram_body,
        grid=grid,
        in_specs=[
            pl.BlockSpec((BM, D), lambda i, j: (i, 0)),
            pl.BlockSpec((BN, D), lambda i, j: (j, 0)),
        ],
        out_specs=pl.BlockSpec((BM, BN), lambda i, j: (i, j)),
        out_shape=jax.ShapeDtypeStruct((N, N), jnp.float32),
    )(z, z)


def kernel(x, edge_index, W, b):
    npad = NROW_P * CH - E
    src_p = jnp.concatenate(
        [edge_index[0].astype(jnp.int32), jnp.full((npad,), N, jnp.int32)])
    dst_p = jnp.concatenate(
        [edge_index[1].astype(jnp.int32), jnp.zeros((npad,), jnp.int32)])
    sd_r = jnp.stack([src_p.reshape(NROW_P, CH), dst_p.reshape(NROW_P, CH)],
                     axis=1)
    dst_r = dst_p.reshape(NROW_P, CH)
    ones_c = jnp.ones((CH,), jnp.float32)
    zeros1 = jnp.zeros((STRIPE,), jnp.float32)
    zeros2 = jnp.zeros((CHK, D), jnp.float32)

    deg0, deg1 = _deg_call(dst_r, ones_c, zeros1)
    g, dinv = _scale_call(deg0.reshape(N, 1), deg1.reshape(N, 1), x)
    s0, s1 = _agg_call(g, sd_r, zeros2)
    z = _z_call(s0, s1, g, dinv, W, b.reshape(1, D))
    return _gram_call(z)


# speedup vs baseline: 1.3387x; 1.3387x over previous
import functools

import jax
import jax.numpy as jnp
from jax import lax
from jax.experimental import pallas as pl
from jax.experimental.pallas import tpu as pltpu
from jax.experimental.pallas import tpu_sc as plsc

N = 10000
D = 64
E = 640000
NC = 2
NS = 16
NW = NC * NS

CH = 128
NROW = E // CH
RPW_P = 160
NROW_P = RPW_P * NW
KSUB = 5
NG = RPW_P // KSUB

N_P = N + 8

CHK = 80
STRIPE = 640
STRIPE_LAST = N - (NS - 1) * STRIPE
assert STRIPE % CHK == 0 and STRIPE_LAST % CHK == 0


def _deg_body(dst_hbm, ones_hbm, zeros1_hbm, deg_out0, deg_out1,
              deg_sh, idx_v, ones_v, zeros_v, zbuf, ssem0, ssem1):
    c = lax.axis_index("c")
    s = lax.axis_index("s")
    wid = c * NS + s

    base = s * STRIPE

    @pl.when(s < NS - 1)
    def _():
        pltpu.sync_copy(zeros1_hbm, zbuf)
        pltpu.sync_copy(zbuf, deg_sh.at[pl.ds(base, STRIPE)])

    @pl.when(s == NS - 1)
    def _():
        pltpu.sync_copy(zeros1_hbm.at[pl.ds(0, STRIPE_LAST)],
                        zbuf.at[pl.ds(0, STRIPE_LAST)])
        pltpu.sync_copy(zbuf.at[pl.ds(0, STRIPE_LAST)],
                        deg_sh.at[pl.ds(base, STRIPE_LAST)])

    pltpu.sync_copy(ones_hbm, ones_v)
    pltpu.sync_copy(zeros1_hbm.at[pl.ds(0, CH)], zeros_v)
    plsc.subcore_barrier()

    row0 = wid * RPW_P
    ssems = (ssem0, ssem1)

    def fire(b, g, ssem):
        r = row0 + g * KSUB
        pltpu.sync_copy(dst_hbm.at[pl.ds(r, KSUB)], idx_v.at[b])
        real = r < NROW

        @pl.when(real)
        def _():
            for j in range(KSUB):
                pltpu.async_copy(ones_v, deg_sh.at[idx_v.at[b, j]], ssem,
                                 add=True)

        @pl.when(jnp.logical_not(real))
        def _():
            for j in range(KSUB):
                pltpu.async_copy(zeros_v, deg_sh.at[idx_v.at[b, j]], ssem,
                                 add=True)

    def drain(b, ssem):
        for j in range(KSUB):
            pltpu.make_async_copy(ones_v, deg_sh.at[idx_v.at[b, j]], ssem).wait()

    def pair(t, carry):
        for b in (0, 1):
            g = 2 * t + b

            @pl.when(t >= 1)
            def _():
                drain(b, ssems[b])

            fire(b, g, ssems[b])
        return carry

    lax.fori_loop(0, NG // 2, pair, 0)
    drain(0, ssem0)
    drain(1, ssem1)
    plsc.subcore_barrier()

    @pl.when(c == 0)
    def _():
        _flush_deg(s, deg_sh, zbuf, deg_out0)

    @pl.when(c == 1)
    def _():
        _flush_deg(s, deg_sh, zbuf, deg_out1)


def _flush_deg(s, deg_sh, zbuf, out_ref):
    base = s * STRIPE

    @pl.when(s < NS - 1)
    def _():
        pltpu.sync_copy(deg_sh.at[pl.ds(base, STRIPE)], zbuf)
        pltpu.sync_copy(zbuf, out_ref.at[pl.ds(base, STRIPE)])

    @pl.when(s == NS - 1)
    def _():
        pltpu.sync_copy(deg_sh.at[pl.ds(base, STRIPE_LAST)],
                        zbuf.at[pl.ds(0, STRIPE_LAST)])
        pltpu.sync_copy(zbuf.at[pl.ds(0, STRIPE_LAST)],
                        out_ref.at[pl.ds(base, STRIPE_LAST)])


def _deg_call(dst_r, ones_c, zeros1):
    mesh = plsc.VectorSubcoreMesh(core_axis_name="c", subcore_axis_name="s")
    return pl.kernel(
        _deg_body,
        out_type=[jax.ShapeDtypeStruct((N,), jnp.float32),
                  jax.ShapeDtypeStruct((N,), jnp.float32)],
        mesh=mesh,
        scratch_types=[
            pltpu.VMEM_SHARED((N,), jnp.float32),
            pltpu.VMEM((2, KSUB, CH), jnp.int32),
            pltpu.VMEM((CH,), jnp.float32),
            pltpu.VMEM((CH,), jnp.float32),
            pltpu.VMEM((STRIPE,), jnp.float32),
            pltpu.SemaphoreType.DMA,
            pltpu.SemaphoreType.DMA,
        ],
        compiler_params=pltpu.CompilerParams(use_tc_tiling_on_sc=False),
    )(dst_r, ones_c, zeros1)


def _agg_body(g_hbm, sd_hbm, zeros2_hbm, s_out0, s_out1,
              s_sh, idx_v, rows_v, zbuf,
              gsem0, gsem1, ssem0, ssem1):
    c = lax.axis_index("c")
    s = lax.axis_index("s")
    wid = c * NS + s

    _stage_table(s, zeros2_hbm, zbuf, s_sh, to_spmem=True)
    plsc.subcore_barrier()

    row0 = wid * RPW_P
    gsems = (gsem0, gsem1)
    ssems = (ssem0, ssem1)

    def load_and_fire_gathers(b, g, gsem):
        r = row0 + g * KSUB
        pltpu.sync_copy(sd_hbm.at[pl.ds(r, KSUB)], idx_v.at[b])
        for j in range(KSUB):
            pltpu.async_copy(g_hbm.at[idx_v.at[b, j, 0]], rows_v.at[b, j], gsem)

    def drain_gathers(b, gsem):
        for j in range(KSUB):
            pltpu.make_async_copy(g_hbm.at[idx_v.at[b, j, 0]],
                                  rows_v.at[b, j], gsem).wait()

    def fire_scatters(b, ssem):
        for j in range(KSUB):
            pltpu.async_copy(rows_v.at[b, j], s_sh.at[idx_v.at[b, j, 1]],
                             ssem, add=True)

    def drain_scatters(b, ssem):
        for j in range(KSUB):
            pltpu.make_async_copy(rows_v.at[b, j],
                                  s_sh.at[idx_v.at[b, j, 1]], ssem).wait()

    load_and_fire_gathers(0, 0, gsem0)

    def pair(t, carry):
        @pl.when(t >= 1)
        def _():
            drain_scatters(1, ssem1)

        load_and_fire_gathers(1, 2 * t + 1, gsem1)
        drain_gathers(0, gsem0)
        fire_scatters(0, ssem0)

        @pl.when(t < NG // 2 - 1)
        def _():
            drain_scatters(0, ssem0)
            load_and_fire_gathers(0, 2 * t + 2, gsem0)

        drain_gathers(1, gsem1)
        fire_scatters(1, ssem1)
        return carry

    lax.fori_loop(0, NG // 2, pair, 0)
    drain_scatters(0, ssem0)
    drain_scatters(1, ssem1)
    plsc.subcore_barrier()

    @pl.when(c == 0)
    def _():
        _stage_table(s, s_out0, zbuf, s_sh, to_spmem=False)

    @pl.when(c == 1)
    def _():
        _stage_table(s, s_out1, zbuf, s_sh, to_spmem=False)


def _stage_table(s, hbm_ref, zbuf, sh_ref, to_spmem):
    base = s * STRIPE

    def move_chunk(off):
        if to_spmem:
            pltpu.sync_copy(hbm_ref, zbuf)
            pltpu.sync_copy(zbuf, sh_ref.at[pl.ds(base + off, CHK)])
        else:
            pltpu.sync_copy(sh_ref.at[pl.ds(base + off, CHK)], zbuf)
            pltpu.sync_copy(zbuf, hbm_ref.at[pl.ds(base + off, CHK)])

    @pl.when(s < NS - 1)
    def _():
        for i in range(STRIPE // CHK):
            move_chunk(i * CHK)

    @pl.when(s == NS - 1)
    def _():
        for i in range(STRIPE_LAST // CHK):
            move_chunk(i * CHK)


def _agg_call(g, sd_r, zeros2):
    mesh = plsc.VectorSubcoreMesh(core_axis_name="c", subcore_axis_name="s")
    return pl.kernel(
        _agg_body,
        out_type=[jax.ShapeDtypeStruct((N, D), jnp.float32),
                  jax.ShapeDtypeStruct((N, D), jnp.float32)],
        mesh=mesh,
        scratch_types=[
            pltpu.VMEM_SHARED((N, D), jnp.float32),
            pltpu.VMEM((2, KSUB, 2, CH), jnp.int32),
            pltpu.VMEM((2, KSUB, CH, D), jnp.float32),
            pltpu.VMEM((CHK, D), jnp.float32),
            pltpu.SemaphoreType.DMA,
            pltpu.SemaphoreType.DMA,
            pltpu.SemaphoreType.DMA,
            pltpu.SemaphoreType.DMA,
        ],
        compiler_params=pltpu.CompilerParams(use_tc_tiling_on_sc=False),
    )(g, sd_r, zeros2)


def _scale_body(deg0_ref, deg1_ref, x_ref, g_ref, dinv_ref):
    total = deg0_ref[...] + deg1_ref[...] + 1.0
    dinv = lax.rsqrt(total)
    dinv_ref[...] = dinv
    g_ref[pl.ds(0, N), :] = x_ref[...] * dinv
    g_ref[pl.ds(N, N_P - N), :] = jnp.zeros((N_P - N, D), jnp.float32)


def _scale_call(deg0, deg1, x):
    return pl.pallas_call(
        _scale_body,
        out_shape=[
            jax.ShapeDtypeStruct((N_P, D), jnp.float32),
            jax.ShapeDtypeStruct((N, 1), jnp.float32),
        ],
    )(deg0, deg1, x)


def _z_body(s0_ref, s1_ref, g_ref, dinv_ref, w_ref, b_ref, z_ref):
    agg = (s0_ref[...] + s1_ref[...] + g_ref[pl.ds(0, N), :]) * dinv_ref[...]
    z = jnp.dot(agg, w_ref[...], preferred_element_type=jnp.float32) + b_ref[...]
    z_ref[...] = jnp.maximum(z, 0.0)


def _z_call(s0, s1, g, dinv, W, b2):
    return pl.pallas_call(
        _z_body,
        out_shape=jax.ShapeDtypeStruct((N, D), jnp.float32),
    )(s0, s1, g, dinv, W, b2)


BM = 2000
BN = 2048


def _gram_body(zi_ref, zj_ref, out_ref):
    out_ref[...] = lax.dot_general(
        zi_ref[...], zj_ref[...],
        (((1,), (1,)), ((), ())),
        preferred_element_type=jnp.float32,
    )


def _gram_call(z):
    grid = (N // BM, (N + BN - 1) // BN)
    return pl.pallas_call(
        _gram_body,
        grid=grid,
        in_specs=[
            pl.BlockSpec((BM, D), lambda i, j: (i, 0)),
            pl.BlockSpec((BN, D), lambda i, j: (j, 0)),
        ],
        out_specs=pl.BlockSpec((BM, BN), lambda i, j: (i, j)),
        out_shape=jax.ShapeDtypeStruct((N, N), jnp.float32),
    )(z, z)


def kernel(x, edge_index, W, b):
    npad = NROW_P * CH - E
    pad_i = jnp.arange(npad, dtype=jnp.int32)
    src_p = jnp.concatenate(
        [edge_index[0].astype(jnp.int32), N + (pad_i % (N_P - N))])
    dst_p = jnp.concatenate(
        [edge_index[1].astype(jnp.int32), (pad_i * 79) % N])
    sd_r = jnp.stack([src_p.reshape(NROW_P, CH), dst_p.reshape(NROW_P, CH)],
                     axis=1)
    dst_r = dst_p.reshape(NROW_P, CH)
    ones_c = jnp.ones((CH,), jnp.float32)
    zeros1 = jnp.zeros((STRIPE,), jnp.float32)
    zeros2 = jnp.zeros((CHK, D), jnp.float32)

    deg0, deg1 = _deg_call(dst_r, ones_c, zeros1)
    g, dinv = _scale_call(deg0.reshape(N, 1), deg1.reshape(N, 1), x)
    s0, s1 = _agg_call(g, sd_r, zeros2)
    z = _z_call(s0, s1, g, dinv, W, b.reshape(1, D))
    return _gram_call(z)


# speedup vs baseline: 1.7417x; 1.3011x over previous
import functools

import jax
import jax.numpy as jnp
from jax import lax
from jax.experimental import pallas as pl
from jax.experimental.pallas import tpu as pltpu
from jax.experimental.pallas import tpu_sc as plsc

N = 10000
D = 64
E = 640000
NC = 2
NS = 16
NW = NC * NS

CH = 128
NROW = E // CH
RPW_P = 160
NROW_P = RPW_P * NW
KSUB = 5
NG = RPW_P // KSUB

N_P = N + 8

CHK = 80
STRIPE = 640
STRIPE_LAST = N - (NS - 1) * STRIPE
assert STRIPE % CHK == 0 and STRIPE_LAST % CHK == 0


def _deg_body(dst_hbm, ones_hbm, zeros1_hbm, deg_out0, deg_out1,
              deg_sh, idx_v, ones_v, zbuf, ssem0, ssem1):
    c = lax.axis_index("c")
    s = lax.axis_index("s")
    wid = c * NS + s

    base = s * STRIPE

    @pl.when(s < NS - 1)
    def _():
        pltpu.sync_copy(zeros1_hbm, zbuf)
        pltpu.sync_copy(zbuf, deg_sh.at[pl.ds(base, STRIPE)])

    @pl.when(s == NS - 1)
    def _():
        pltpu.sync_copy(zeros1_hbm.at[pl.ds(0, STRIPE_LAST)],
                        zbuf.at[pl.ds(0, STRIPE_LAST)])
        pltpu.sync_copy(zbuf.at[pl.ds(0, STRIPE_LAST)],
                        deg_sh.at[pl.ds(base, STRIPE_LAST)])

    pltpu.sync_copy(ones_hbm, ones_v)
    plsc.subcore_barrier()

    row0 = wid * RPW_P
    ssems = (ssem0, ssem1)

    def fire(b, g, ssem):
        r = row0 + g * KSUB

        @pl.when(r < NROW)
        def _():
            pltpu.sync_copy(dst_hbm.at[pl.ds(r, KSUB)], idx_v.at[b])
            for j in range(KSUB):
                pltpu.async_copy(ones_v, deg_sh.at[idx_v.at[b, j]], ssem,
                                 add=True)

    def drain(b, g, ssem):
        @pl.when(row0 + g * KSUB < NROW)
        def _():
            for j in range(KSUB):
                pltpu.make_async_copy(ones_v, deg_sh.at[idx_v.at[b, j]],
                                      ssem).wait()

    def pair(t, carry):
        for b in (0, 1):
            g = 2 * t + b

            @pl.when(t >= 1)
            def _():
                drain(b, g - 2, ssems[b])

            fire(b, g, ssems[b])
        return carry

    lax.fori_loop(0, NG // 2, pair, 0)
    drain(0, NG - 2, ssem0)
    drain(1, NG - 1, ssem1)
    plsc.subcore_barrier()

    @pl.when(c == 0)
    def _():
        _flush_deg(s, deg_sh, zbuf, deg_out0)

    @pl.when(c == 1)
    def _():
        _flush_deg(s, deg_sh, zbuf, deg_out1)


def _flush_deg(s, deg_sh, zbuf, out_ref):
    base = s * STRIPE

    @pl.when(s < NS - 1)
    def _():
        pltpu.sync_copy(deg_sh.at[pl.ds(base, STRIPE)], zbuf)
        pltpu.sync_copy(zbuf, out_ref.at[pl.ds(base, STRIPE)])

    @pl.when(s == NS - 1)
    def _():
        pltpu.sync_copy(deg_sh.at[pl.ds(base, STRIPE_LAST)],
                        zbuf.at[pl.ds(0, STRIPE_LAST)])
        pltpu.sync_copy(zbuf.at[pl.ds(0, STRIPE_LAST)],
                        out_ref.at[pl.ds(base, STRIPE_LAST)])


def _deg_call(dst_r, ones_c, zeros1):
    mesh = plsc.VectorSubcoreMesh(core_axis_name="c", subcore_axis_name="s")
    return pl.kernel(
        _deg_body,
        out_type=[jax.ShapeDtypeStruct((N,), jnp.float32),
                  jax.ShapeDtypeStruct((N,), jnp.float32)],
        mesh=mesh,
        scratch_types=[
            pltpu.VMEM_SHARED((N,), jnp.float32),
            pltpu.VMEM((2, KSUB, CH), jnp.int32),
            pltpu.VMEM((CH,), jnp.float32),
            pltpu.VMEM((STRIPE,), jnp.float32),
            pltpu.SemaphoreType.DMA,
            pltpu.SemaphoreType.DMA,
        ],
        compiler_params=pltpu.CompilerParams(use_tc_tiling_on_sc=False),
    )(dst_r, ones_c, zeros1)


def _agg_body(g_hbm, sd_hbm, zeros2_hbm, s_out0, s_out1,
              s_sh, idx_v, rows_v, zbuf,
              gsem0, gsem1, ssem0, ssem1):
    c = lax.axis_index("c")
    s = lax.axis_index("s")
    wid = c * NS + s

    _stage_table(s, zeros2_hbm, zbuf, s_sh, to_spmem=True)
    plsc.subcore_barrier()

    row0 = wid * RPW_P
    gsems = (gsem0, gsem1)
    ssems = (ssem0, ssem1)

    def real(g):
        return row0 + g * KSUB < NROW

    def load_and_fire_gathers(b, g, gsem):
        r = row0 + g * KSUB

        @pl.when(real(g))
        def _():
            pltpu.sync_copy(sd_hbm.at[pl.ds(r, KSUB)], idx_v.at[b])
            for j in range(KSUB):
                pltpu.async_copy(g_hbm.at[idx_v.at[b, j, 0]], rows_v.at[b, j],
                                 gsem)

    def drain_gathers(b, g, gsem):
        @pl.when(real(g))
        def _():
            for j in range(KSUB):
                pltpu.make_async_copy(g_hbm.at[idx_v.at[b, j, 0]],
                                      rows_v.at[b, j], gsem).wait()

    def fire_scatters(b, g, ssem):
        @pl.when(real(g))
        def _():
            for j in range(KSUB):
                pltpu.async_copy(rows_v.at[b, j], s_sh.at[idx_v.at[b, j, 1]],
                                 ssem, add=True)

    def drain_scatters(b, g, ssem):
        @pl.when(real(g))
        def _():
            for j in range(KSUB):
                pltpu.make_async_copy(rows_v.at[b, j],
                                      s_sh.at[idx_v.at[b, j, 1]], ssem).wait()

    load_and_fire_gathers(0, 0, gsem0)

    def pair(t, carry):
        @pl.when(t >= 1)
        def _():
            drain_scatters(1, 2 * t - 1, ssem1)

        load_and_fire_gathers(1, 2 * t + 1, gsem1)
        drain_gathers(0, 2 * t, gsem0)
        fire_scatters(0, 2 * t, ssem0)

        @pl.when(t < NG // 2 - 1)
        def _():
            drain_scatters(0, 2 * t, ssem0)
            load_and_fire_gathers(0, 2 * t + 2, gsem0)

        drain_gathers(1, 2 * t + 1, gsem1)
        fire_scatters(1, 2 * t + 1, ssem1)
        return carry

    lax.fori_loop(0, NG // 2, pair, 0)
    drain_scatters(0, NG - 2, ssem0)
    drain_scatters(1, NG - 1, ssem1)
    plsc.subcore_barrier()

    @pl.when(c == 0)
    def _():
        _stage_table(s, s_out0, zbuf, s_sh, to_spmem=False)

    @pl.when(c == 1)
    def _():
        _stage_table(s, s_out1, zbuf, s_sh, to_spmem=False)


def _stage_table(s, hbm_ref, zbuf, sh_ref, to_spmem):
    base = s * STRIPE

    def move_chunk(off):
        if to_spmem:
            pltpu.sync_copy(hbm_ref, zbuf)
            pltpu.sync_copy(zbuf, sh_ref.at[pl.ds(base + off, CHK)])
        else:
            pltpu.sync_copy(sh_ref.at[pl.ds(base + off, CHK)], zbuf)
            pltpu.sync_copy(zbuf, hbm_ref.at[pl.ds(base + off, CHK)])

    @pl.when(s < NS - 1)
    def _():
        for i in range(STRIPE // CHK):
            move_chunk(i * CHK)

    @pl.when(s == NS - 1)
    def _():
        for i in range(STRIPE_LAST // CHK):
            move_chunk(i * CHK)


def _agg_call(g, sd_r, zeros2):
    mesh = plsc.VectorSubcoreMesh(core_axis_name="c", subcore_axis_name="s")
    return pl.kernel(
        _agg_body,
        out_type=[jax.ShapeDtypeStruct((N, D), jnp.float32),
                  jax.ShapeDtypeStruct((N, D), jnp.float32)],
        mesh=mesh,
        scratch_types=[
            pltpu.VMEM_SHARED((N, D), jnp.float32),
            pltpu.VMEM((2, KSUB, 2, CH), jnp.int32),
            pltpu.VMEM((2, KSUB, CH, D), jnp.float32),
            pltpu.VMEM((CHK, D), jnp.float32),
            pltpu.SemaphoreType.DMA,
            pltpu.SemaphoreType.DMA,
            pltpu.SemaphoreType.DMA,
            pltpu.SemaphoreType.DMA,
        ],
        compiler_params=pltpu.CompilerParams(use_tc_tiling_on_sc=False),
    )(g, sd_r, zeros2)


def _scale_body(deg0_ref, deg1_ref, x_ref, g_ref, dinv_ref):
    total = deg0_ref[...] + deg1_ref[...] + 1.0
    dinv = lax.rsqrt(total)
    dinv_ref[...] = dinv
    g_ref[pl.ds(0, N), :] = x_ref[...] * dinv
    g_ref[pl.ds(N, N_P - N), :] = jnp.zeros((N_P - N, D), jnp.float32)


def _scale_call(deg0, deg1, x):
    return pl.pallas_call(
        _scale_body,
        out_shape=[
            jax.ShapeDtypeStruct((N_P, D), jnp.float32),
            jax.ShapeDtypeStruct((N, 1), jnp.float32),
        ],
    )(deg0, deg1, x)


def _z_body(s0_ref, s1_ref, g_ref, dinv_ref, w_ref, b_ref, z_ref):
    agg = (s0_ref[...] + s1_ref[...] + g_ref[pl.ds(0, N), :]) * dinv_ref[...]
    z = jnp.dot(agg, w_ref[...], preferred_element_type=jnp.float32) + b_ref[...]
    z_ref[...] = jnp.maximum(z, 0.0)


def _z_call(s0, s1, g, dinv, W, b2):
    return pl.pallas_call(
        _z_body,
        out_shape=jax.ShapeDtypeStruct((N, D), jnp.float32),
    )(s0, s1, g, dinv, W, b2)


BM = 2000
BN = 2048


def _gram_body(zi_ref, zj_ref, out_ref):
    out_ref[...] = lax.dot_general(
        zi_ref[...], zj_ref[...],
        (((1,), (1,)), ((), ())),
        preferred_element_type=jnp.float32,
    )


def _gram_call(z):
    grid = (N // BM, (N + BN - 1) // BN)
    return pl.pallas_call(
        _gram_body,
        grid=grid,
        in_specs=[
            pl.BlockSpec((BM, D), lambda i, j: (i, 0)),
            pl.BlockSpec((BN, D), lambda i, j: (j, 0)),
        ],
        out_specs=pl.BlockSpec((BM, BN), lambda i, j: (i, j)),
        out_shape=jax.ShapeDtypeStruct((N, N), jnp.float32),
    )(z, z)


def kernel(x, edge_index, W, b):
    npad = NROW_P * CH - E
    pad_i = jnp.arange(npad, dtype=jnp.int32)
    src_p = jnp.concatenate(
        [edge_index[0].astype(jnp.int32), N + (pad_i % (N_P - N))])
    dst_p = jnp.concatenate(
        [edge_index[1].astype(jnp.int32), (pad_i * 79) % N])
    sd_r = jnp.stack([src_p.reshape(NROW_P, CH), dst_p.reshape(NROW_P, CH)],
                     axis=1)
    dst_r = dst_p.reshape(NROW_P, CH)
    ones_c = jnp.ones((CH,), jnp.float32)
    zeros1 = jnp.zeros((STRIPE,), jnp.float32)
    zeros2 = jnp.zeros((CHK, D), jnp.float32)

    deg0, deg1 = _deg_call(dst_r, ones_c, zeros1)
    g, dinv = _scale_call(deg0.reshape(N, 1), deg1.reshape(N, 1), x)
    s0, s1 = _agg_call(g, sd_r, zeros2)
    z = _z_call(s0, s1, g, dinv, W, b.reshape(1, D))
    return _gram_call(z)


# speedup vs baseline: 1.8544x; 1.0647x over previous
import functools

import jax
import jax.numpy as jnp
from jax import lax
from jax.experimental import pallas as pl
from jax.experimental.pallas import tpu as pltpu
from jax.experimental.pallas import tpu_sc as plsc

N = 10000
D = 64
E = 640000
NC = 2
NS = 16
NW = NC * NS

CH = 128
NROW = E // CH
RPW_P = 160
NROW_P = RPW_P * NW
KSUB = 5
NG = RPW_P // KSUB

CHK = 80
STRIPE = 640
STRIPE_LAST = N - (NS - 1) * STRIPE
assert STRIPE % CHK == 0 and STRIPE_LAST % CHK == 0


def _deg_body(dst_hbm, ones_hbm, zeros1_hbm, deg_out0, deg_out1,
              deg_sh, idx_v, ones_v, zbuf, ssem0, ssem1):
    c = lax.axis_index("c")
    s = lax.axis_index("s")
    wid = c * NS + s

    base = s * STRIPE

    @pl.when(s < NS - 1)
    def _():
        pltpu.sync_copy(zeros1_hbm, zbuf)
        pltpu.sync_copy(zbuf, deg_sh.at[pl.ds(base, STRIPE)])

    @pl.when(s == NS - 1)
    def _():
        pltpu.sync_copy(zeros1_hbm.at[pl.ds(0, STRIPE_LAST)],
                        zbuf.at[pl.ds(0, STRIPE_LAST)])
        pltpu.sync_copy(zbuf.at[pl.ds(0, STRIPE_LAST)],
                        deg_sh.at[pl.ds(base, STRIPE_LAST)])

    pltpu.sync_copy(ones_hbm, ones_v)
    plsc.subcore_barrier()

    row0 = wid * RPW_P
    ssems = (ssem0, ssem1)

    def fire(b, g, ssem):
        r = row0 + g * KSUB

        @pl.when(r < NROW)
        def _():
            pltpu.sync_copy(dst_hbm.at[pl.ds(r, KSUB)], idx_v.at[b])
            for j in range(KSUB):
                pltpu.async_copy(ones_v, deg_sh.at[idx_v.at[b, j]], ssem,
                                 add=True)

    def drain(b, g, ssem):
        @pl.when(row0 + g * KSUB < NROW)
        def _():
            for j in range(KSUB):
                pltpu.make_async_copy(ones_v, deg_sh.at[idx_v.at[b, j]],
                                      ssem).wait()

    def pair(t, carry):
        for b in (0, 1):
            g = 2 * t + b

            @pl.when(t >= 1)
            def _():
                drain(b, g - 2, ssems[b])

            fire(b, g, ssems[b])
        return carry

    lax.fori_loop(0, NG // 2, pair, 0)
    drain(0, NG - 2, ssem0)
    drain(1, NG - 1, ssem1)
    plsc.subcore_barrier()

    @pl.when(c == 0)
    def _():
        _flush_deg(s, deg_sh, zbuf, deg_out0)

    @pl.when(c == 1)
    def _():
        _flush_deg(s, deg_sh, zbuf, deg_out1)


def _flush_deg(s, deg_sh, zbuf, out_ref):
    base = s * STRIPE

    @pl.when(s < NS - 1)
    def _():
        pltpu.sync_copy(deg_sh.at[pl.ds(base, STRIPE)], zbuf)
        pltpu.sync_copy(zbuf, out_ref.at[pl.ds(base, STRIPE)])

    @pl.when(s == NS - 1)
    def _():
        pltpu.sync_copy(deg_sh.at[pl.ds(base, STRIPE_LAST)],
                        zbuf.at[pl.ds(0, STRIPE_LAST)])
        pltpu.sync_copy(zbuf.at[pl.ds(0, STRIPE_LAST)],
                        out_ref.at[pl.ds(base, STRIPE_LAST)])


def _deg_call(dst_r, ones_c, zeros1):
    mesh = plsc.VectorSubcoreMesh(core_axis_name="c", subcore_axis_name="s")
    return pl.kernel(
        _deg_body,
        out_type=[jax.ShapeDtypeStruct((N,), jnp.float32),
                  jax.ShapeDtypeStruct((N,), jnp.float32)],
        mesh=mesh,
        scratch_types=[
            pltpu.VMEM_SHARED((N,), jnp.float32),
            pltpu.VMEM((2, KSUB, CH), jnp.int32),
            pltpu.VMEM((CH,), jnp.float32),
            pltpu.VMEM((STRIPE,), jnp.float32),
            pltpu.SemaphoreType.DMA,
            pltpu.SemaphoreType.DMA,
        ],
        compiler_params=pltpu.CompilerParams(use_tc_tiling_on_sc=False),
    )(dst_r, ones_c, zeros1)


def _agg_body(g_hbm, sd_hbm, zeros2_hbm, s_out0, s_out1,
              s_sh, idx_v, rows_v, zbuf,
              gsem0, gsem1, ssem0, ssem1):
    c = lax.axis_index("c")
    s = lax.axis_index("s")
    wid = c * NS + s

    _stage_table(s, zeros2_hbm, zbuf, s_sh, to_spmem=True)
    plsc.subcore_barrier()

    row0 = wid * RPW_P
    gsems = (gsem0, gsem1)
    ssems = (ssem0, ssem1)

    def real(g):
        return row0 + g * KSUB < NROW

    def load_and_fire_gathers(b, g, gsem):
        r = row0 + g * KSUB

        @pl.when(real(g))
        def _():
            pltpu.sync_copy(sd_hbm.at[pl.ds(r, KSUB)], idx_v.at[b])
            for j in range(KSUB):
                pltpu.async_copy(g_hbm.at[idx_v.at[b, j, 0]], rows_v.at[b, j],
                                 gsem)

    def drain_gathers(b, g, gsem):
        @pl.when(real(g))
        def _():
            for j in range(KSUB):
                pltpu.make_async_copy(g_hbm.at[idx_v.at[b, j, 0]],
                                      rows_v.at[b, j], gsem).wait()

    def fire_scatters(b, g, ssem):
        @pl.when(real(g))
        def _():
            for j in range(KSUB):
                pltpu.async_copy(rows_v.at[b, j], s_sh.at[idx_v.at[b, j, 1]],
                                 ssem, add=True)

    def drain_scatters(b, g, ssem):
        @pl.when(real(g))
        def _():
            for j in range(KSUB):
                pltpu.make_async_copy(rows_v.at[b, j],
                                      s_sh.at[idx_v.at[b, j, 1]], ssem).wait()

    load_and_fire_gathers(0, 0, gsem0)

    def pair(t, carry):
        @pl.when(t >= 1)
        def _():
            drain_scatters(1, 2 * t - 1, ssem1)

        load_and_fire_gathers(1, 2 * t + 1, gsem1)
        drain_gathers(0, 2 * t, gsem0)
        fire_scatters(0, 2 * t, ssem0)

        @pl.when(t < NG // 2 - 1)
        def _():
            drain_scatters(0, 2 * t, ssem0)
            load_and_fire_gathers(0, 2 * t + 2, gsem0)

        drain_gathers(1, 2 * t + 1, gsem1)
        fire_scatters(1, 2 * t + 1, ssem1)
        return carry

    lax.fori_loop(0, NG // 2, pair, 0)
    drain_scatters(0, NG - 2, ssem0)
    drain_scatters(1, NG - 1, ssem1)
    plsc.subcore_barrier()

    @pl.when(c == 0)
    def _():
        _stage_table(s, s_out0, zbuf, s_sh, to_spmem=False)

    @pl.when(c == 1)
    def _():
        _stage_table(s, s_out1, zbuf, s_sh, to_spmem=False)


def _stage_table(s, hbm_ref, zbuf, sh_ref, to_spmem):
    base = s * STRIPE

    def move_chunk(off):
        if to_spmem:
            pltpu.sync_copy(hbm_ref, zbuf)
            pltpu.sync_copy(zbuf, sh_ref.at[pl.ds(base + off, CHK)])
        else:
            pltpu.sync_copy(sh_ref.at[pl.ds(base + off, CHK)], zbuf)
            pltpu.sync_copy(zbuf, hbm_ref.at[pl.ds(base + off, CHK)])

    @pl.when(s < NS - 1)
    def _():
        for i in range(STRIPE // CHK):
            move_chunk(i * CHK)

    @pl.when(s == NS - 1)
    def _():
        for i in range(STRIPE_LAST // CHK):
            move_chunk(i * CHK)


def _agg_call(g, sd_r, zeros2):
    mesh = plsc.VectorSubcoreMesh(core_axis_name="c", subcore_axis_name="s")
    return pl.kernel(
        _agg_body,
        out_type=[jax.ShapeDtypeStruct((N, D), jnp.float32),
                  jax.ShapeDtypeStruct((N, D), jnp.float32)],
        mesh=mesh,
        scratch_types=[
            pltpu.VMEM_SHARED((N, D), jnp.float32),
            pltpu.VMEM((2, KSUB, 2, CH), jnp.int32),
            pltpu.VMEM((2, KSUB, CH, D), jnp.float32),
            pltpu.VMEM((CHK, D), jnp.float32),
            pltpu.SemaphoreType.DMA,
            pltpu.SemaphoreType.DMA,
            pltpu.SemaphoreType.DMA,
            pltpu.SemaphoreType.DMA,
        ],
        compiler_params=pltpu.CompilerParams(use_tc_tiling_on_sc=False),
    )(g, sd_r, zeros2)


def _scale_body(deg0_ref, deg1_ref, x_ref, g_ref, dinv_ref):
    total = deg0_ref[...] + deg1_ref[...] + 1.0
    dinv = lax.rsqrt(total)
    dinv_ref[...] = dinv
    g_ref[...] = x_ref[...] * dinv


def _scale_call(deg0, deg1, x):
    return pl.pallas_call(
        _scale_body,
        out_shape=[
            jax.ShapeDtypeStruct((N, D), jnp.float32),
            jax.ShapeDtypeStruct((N, 1), jnp.float32),
        ],
    )(deg0, deg1, x)


BM = 2000
BN = 2048
ZROWS = ((N + BN - 1) // BN) * BN


def _gram_body(s0_ref, s1_ref, g_ref, dinv_ref, w_ref, b_ref, out_ref, z_scr):
    i = pl.program_id(0)
    j = pl.program_id(1)

    @pl.when(jnp.logical_and(i == 0, j == 0))
    def _():
        agg = (s0_ref[...] + s1_ref[...] + g_ref[...]) * dinv_ref[...]
        z = jnp.dot(agg, w_ref[...],
                    preferred_element_type=jnp.float32) + b_ref[...]
        z_scr[pl.ds(0, N), :] = jnp.maximum(z, 0.0)
        z_scr[pl.ds(N, ZROWS - N), :] = jnp.zeros((ZROWS - N, D), jnp.float32)

    zi = z_scr[pl.ds(i * BM, BM), :]
    zj = z_scr[pl.ds(j * BN, BN), :]
    out_ref[...] = lax.dot_general(
        zi, zj,
        (((1,), (1,)), ((), ())),
        preferred_element_type=jnp.float32,
    )


def _gram_call(s0, s1, g, dinv, W, b2):
    grid = (N // BM, (N + BN - 1) // BN)
    full = lambda i, j: (0, 0)
    return pl.pallas_call(
        _gram_body,
        grid=grid,
        in_specs=[
            pl.BlockSpec((N, D), full),
            pl.BlockSpec((N, D), full),
            pl.BlockSpec((N, D), full),
            pl.BlockSpec((N, 1), full),
            pl.BlockSpec((D, D), full),
            pl.BlockSpec((1, D), full),
        ],
        out_specs=pl.BlockSpec((BM, BN), lambda i, j: (i, j)),
        out_shape=jax.ShapeDtypeStruct((N, N), jnp.float32),
        scratch_shapes=[pltpu.VMEM((ZROWS, D), jnp.float32)],
    )(s0, s1, g, dinv, W, b2)


def kernel(x, edge_index, W, b):
    src_r = edge_index[0].astype(jnp.int32).reshape(NROW, CH)
    dst_r = edge_index[1].astype(jnp.int32).reshape(NROW, CH)
    sd_r = jnp.stack([src_r, dst_r], axis=1)
    ones_c = jnp.ones((CH,), jnp.float32)
    zeros1 = jnp.zeros((STRIPE,), jnp.float32)
    zeros2 = jnp.zeros((CHK, D), jnp.float32)

    deg0, deg1 = _deg_call(dst_r, ones_c, zeros1)
    g, dinv = _scale_call(deg0.reshape(N, 1), deg1.reshape(N, 1), x)
    s0, s1 = _agg_call(g, sd_r, zeros2)
    return _gram_call(s0, s1, g, dinv, W, b.reshape(1, D))


# speedup vs baseline: 1.9206x; 1.0357x over previous
import functools

import jax
import jax.numpy as jnp
from jax import lax
from jax.experimental import pallas as pl
from jax.experimental.pallas import tpu as pltpu
from jax.experimental.pallas import tpu_sc as plsc

N = 10000
D = 64
E = 640000
NC = 2
NS = 16
NW = NC * NS

CH = 128
NROW = E // CH
RPW_P = 160
NROW_P = RPW_P * NW
KSUB = 5
NG = RPW_P // KSUB
RPW_LAST = NROW - (NW - 1) * RPW_P

CHK = 80
STRIPE = 640
STRIPE_LAST = N - (NS - 1) * STRIPE
assert STRIPE % CHK == 0 and STRIPE_LAST % CHK == 0


def _deg_body(dst_hbm, ones_hbm, zeros1_hbm, deg_out0, deg_out1,
              deg_sh, idx_v, ones_v, zbuf, ssem0, ssem1):
    c = lax.axis_index("c")
    s = lax.axis_index("s")
    wid = c * NS + s

    base = s * STRIPE

    @pl.when(s < NS - 1)
    def _():
        pltpu.sync_copy(zeros1_hbm, zbuf)
        pltpu.sync_copy(zbuf, deg_sh.at[pl.ds(base, STRIPE)])

    @pl.when(s == NS - 1)
    def _():
        pltpu.sync_copy(zeros1_hbm.at[pl.ds(0, STRIPE_LAST)],
                        zbuf.at[pl.ds(0, STRIPE_LAST)])
        pltpu.sync_copy(zbuf.at[pl.ds(0, STRIPE_LAST)],
                        deg_sh.at[pl.ds(base, STRIPE_LAST)])

    pltpu.sync_copy(ones_hbm, ones_v)
    plsc.subcore_barrier()

    row0 = wid * RPW_P
    ssems = (ssem0, ssem1)

    @pl.when(wid < NW - 1)
    def _():
        pltpu.sync_copy(dst_hbm.at[pl.ds(row0, RPW_P)], idx_v)

    @pl.when(wid == NW - 1)
    def _():
        pltpu.sync_copy(dst_hbm.at[pl.ds(row0, RPW_LAST)],
                        idx_v.at[pl.ds(0, RPW_LAST)])

    def fire(g, ssem):
        @pl.when(row0 + g * KSUB < NROW)
        def _():
            for j in range(KSUB):
                pltpu.async_copy(ones_v, deg_sh.at[idx_v.at[g * KSUB + j]],
                                 ssem, add=True)

    def drain(g, ssem):
        @pl.when(row0 + g * KSUB < NROW)
        def _():
            for j in range(KSUB):
                pltpu.make_async_copy(ones_v, deg_sh.at[idx_v.at[g * KSUB + j]],
                                      ssem).wait()

    def pair(t, carry):
        for b in (0, 1):
            g = 2 * t + b

            @pl.when(t >= 1)
            def _():
                drain(g - 2, ssems[b])

            fire(g, ssems[b])
        return carry

    lax.fori_loop(0, NG // 2, pair, 0)
    drain(NG - 2, ssem0)
    drain(NG - 1, ssem1)
    plsc.subcore_barrier()

    @pl.when(c == 0)
    def _():
        _flush_deg(s, deg_sh, zbuf, deg_out0)

    @pl.when(c == 1)
    def _():
        _flush_deg(s, deg_sh, zbuf, deg_out1)


def _flush_deg(s, deg_sh, zbuf, out_ref):
    base = s * STRIPE

    @pl.when(s < NS - 1)
    def _():
        pltpu.sync_copy(deg_sh.at[pl.ds(base, STRIPE)], zbuf)
        pltpu.sync_copy(zbuf, out_ref.at[pl.ds(base, STRIPE)])

    @pl.when(s == NS - 1)
    def _():
        pltpu.sync_copy(deg_sh.at[pl.ds(base, STRIPE_LAST)],
                        zbuf.at[pl.ds(0, STRIPE_LAST)])
        pltpu.sync_copy(zbuf.at[pl.ds(0, STRIPE_LAST)],
                        out_ref.at[pl.ds(base, STRIPE_LAST)])


def _deg_call(dst_r, ones_c, zeros1):
    mesh = plsc.VectorSubcoreMesh(core_axis_name="c", subcore_axis_name="s")
    return pl.kernel(
        _deg_body,
        out_type=[jax.ShapeDtypeStruct((N,), jnp.float32),
                  jax.ShapeDtypeStruct((N,), jnp.float32)],
        mesh=mesh,
        scratch_types=[
            pltpu.VMEM_SHARED((N,), jnp.float32),
            pltpu.VMEM((RPW_P, CH), jnp.int32),
            pltpu.VMEM((CH,), jnp.float32),
            pltpu.VMEM((STRIPE,), jnp.float32),
            pltpu.SemaphoreType.DMA,
            pltpu.SemaphoreType.DMA,
        ],
        compiler_params=pltpu.CompilerParams(use_tc_tiling_on_sc=False),
    )(dst_r, ones_c, zeros1)


def _agg_body(g_hbm, sd_hbm, zeros2_hbm, s_out0, s_out1,
              s_sh, idx_v, rows_v, zbuf,
              gsem0, gsem1, ssem0, ssem1):
    c = lax.axis_index("c")
    s = lax.axis_index("s")
    wid = c * NS + s

    _stage_table(s, zeros2_hbm, zbuf, s_sh, to_spmem=True)
    plsc.subcore_barrier()

    row0 = wid * RPW_P
    gsems = (gsem0, gsem1)
    ssems = (ssem0, ssem1)

    def real(g):
        return row0 + g * KSUB < NROW

    def load_and_fire_gathers(b, g, gsem):
        r = row0 + g * KSUB

        @pl.when(real(g))
        def _():
            pltpu.sync_copy(sd_hbm.at[pl.ds(r, KSUB)], idx_v.at[b])
            for j in range(KSUB):
                pltpu.async_copy(g_hbm.at[idx_v.at[b, j, 0]], rows_v.at[b, j],
                                 gsem)

    def drain_gathers(b, g, gsem):
        @pl.when(real(g))
        def _():
            for j in range(KSUB):
                pltpu.make_async_copy(g_hbm.at[idx_v.at[b, j, 0]],
                                      rows_v.at[b, j], gsem).wait()

    def fire_scatters(b, g, ssem):
        @pl.when(real(g))
        def _():
            for j in range(KSUB):
                pltpu.async_copy(rows_v.at[b, j], s_sh.at[idx_v.at[b, j, 1]],
                                 ssem, add=True)

    def drain_scatters(b, g, ssem):
        @pl.when(real(g))
        def _():
            for j in range(KSUB):
                pltpu.make_async_copy(rows_v.at[b, j],
                                      s_sh.at[idx_v.at[b, j, 1]], ssem).wait()

    load_and_fire_gathers(0, 0, gsem0)

    def pair(t, carry):
        @pl.when(t >= 1)
        def _():
            drain_scatters(1, 2 * t - 1, ssem1)

        load_and_fire_gathers(1, 2 * t + 1, gsem1)
        drain_gathers(0, 2 * t, gsem0)
        fire_scatters(0, 2 * t, ssem0)

        @pl.when(t < NG // 2 - 1)
        def _():
            drain_scatters(0, 2 * t, ssem0)
            load_and_fire_gathers(0, 2 * t + 2, gsem0)

        drain_gathers(1, 2 * t + 1, gsem1)
        fire_scatters(1, 2 * t + 1, ssem1)
        return carry

    lax.fori_loop(0, NG // 2, pair, 0)
    drain_scatters(0, NG - 2, ssem0)
    drain_scatters(1, NG - 1, ssem1)
    plsc.subcore_barrier()

    @pl.when(c == 0)
    def _():
        _stage_table(s, s_out0, zbuf, s_sh, to_spmem=False)

    @pl.when(c == 1)
    def _():
        _stage_table(s, s_out1, zbuf, s_sh, to_spmem=False)


def _stage_table(s, hbm_ref, zbuf, sh_ref, to_spmem):
    base = s * STRIPE

    def move_chunk(off):
        if to_spmem:
            pltpu.sync_copy(hbm_ref, zbuf)
            pltpu.sync_copy(zbuf, sh_ref.at[pl.ds(base + off, CHK)])
        else:
            pltpu.sync_copy(sh_ref.at[pl.ds(base + off, CHK)], zbuf)
            pltpu.sync_copy(zbuf, hbm_ref.at[pl.ds(base + off, CHK)])

    @pl.when(s < NS - 1)
    def _():
        for i in range(STRIPE // CHK):
            move_chunk(i * CHK)

    @pl.when(s == NS - 1)
    def _():
        for i in range(STRIPE_LAST // CHK):
            move_chunk(i * CHK)


def _agg_call(g, sd_r, zeros2):
    mesh = plsc.VectorSubcoreMesh(core_axis_name="c", subcore_axis_name="s")
    return pl.kernel(
        _agg_body,
        out_type=[jax.ShapeDtypeStruct((N, D), jnp.float32),
                  jax.ShapeDtypeStruct((N, D), jnp.float32)],
        mesh=mesh,
        scratch_types=[
            pltpu.VMEM_SHARED((N, D), jnp.float32),
            pltpu.VMEM((2, KSUB, 2, CH), jnp.int32),
            pltpu.VMEM((2, KSUB, CH, D), jnp.float32),
            pltpu.VMEM((CHK, D), jnp.float32),
            pltpu.SemaphoreType.DMA,
            pltpu.SemaphoreType.DMA,
            pltpu.SemaphoreType.DMA,
            pltpu.SemaphoreType.DMA,
        ],
        compiler_params=pltpu.CompilerParams(use_tc_tiling_on_sc=False),
    )(g, sd_r, zeros2)


def _scale_body(deg0_ref, deg1_ref, x_ref, g_ref, dinv_ref):
    total = deg0_ref[...] + deg1_ref[...] + 1.0
    dinv = lax.rsqrt(total)
    dinv_ref[...] = dinv
    g_ref[...] = x_ref[...] * dinv


def _scale_call(deg0, deg1, x):
    return pl.pallas_call(
        _scale_body,
        out_shape=[
            jax.ShapeDtypeStruct((N, D), jnp.float32),
            jax.ShapeDtypeStruct((N, 1), jnp.float32),
        ],
    )(deg0, deg1, x)


BM = 2000
BN = 2048
ZROWS = ((N + BN - 1) // BN) * BN


def _gram_body(s0_ref, s1_ref, g_ref, dinv_ref, w_ref, b_ref, out_ref, z_scr):
    i = pl.program_id(0)
    j = pl.program_id(1)

    @pl.when(jnp.logical_and(i == 0, j == 0))
    def _():
        agg = (s0_ref[...] + s1_ref[...] + g_ref[...]) * dinv_ref[...]
        z = jnp.dot(agg, w_ref[...],
                    preferred_element_type=jnp.float32) + b_ref[...]
        z_scr[pl.ds(0, N), :] = jnp.maximum(z, 0.0)
        z_scr[pl.ds(N, ZROWS - N), :] = jnp.zeros((ZROWS - N, D), jnp.float32)

    zi = z_scr[pl.ds(i * BM, BM), :]
    zj = z_scr[pl.ds(j * BN, BN), :]
    out_ref[...] = lax.dot_general(
        zi, zj,
        (((1,), (1,)), ((), ())),
        preferred_element_type=jnp.float32,
    )


def _gram_call(s0, s1, g, dinv, W, b2):
    grid = (N // BM, (N + BN - 1) // BN)
    full = lambda i, j: (0, 0)
    return pl.pallas_call(
        _gram_body,
        grid=grid,
        in_specs=[
            pl.BlockSpec((N, D), full),
            pl.BlockSpec((N, D), full),
            pl.BlockSpec((N, D), full),
            pl.BlockSpec((N, 1), full),
            pl.BlockSpec((D, D), full),
            pl.BlockSpec((1, D), full),
        ],
        out_specs=pl.BlockSpec((BM, BN), lambda i, j: (i, j)),
        out_shape=jax.ShapeDtypeStruct((N, N), jnp.float32),
        scratch_shapes=[pltpu.VMEM((ZROWS, D), jnp.float32)],
    )(s0, s1, g, dinv, W, b2)


def kernel(x, edge_index, W, b):
    src_r = edge_index[0].astype(jnp.int32).reshape(NROW, CH)
    dst_r = edge_index[1].astype(jnp.int32).reshape(NROW, CH)
    sd_r = jnp.stack([src_r, dst_r], axis=1)
    ones_c = jnp.ones((CH,), jnp.float32)
    zeros1 = jnp.zeros((STRIPE,), jnp.float32)
    zeros2 = jnp.zeros((CHK, D), jnp.float32)

    deg0, deg1 = _deg_call(dst_r, ones_c, zeros1)
    g, dinv = _scale_call(deg0.reshape(N, 1), deg1.reshape(N, 1), x)
    s0, s1 = _agg_call(g, sd_r, zeros2)
    return _gram_call(s0, s1, g, dinv, W, b.reshape(1, D))
